# baseline (speedup 1.0000x reference)
import functools

import jax
import jax.numpy as jnp
from jax import lax
from jax.experimental import pallas as pl
from jax.experimental.pallas import tpu as pltpu
from jax.experimental.pallas import tpu_sc as plsc

N_NODES_C = 10000
N_EDGES_C = 320000
EMB_C = 128

NC = 2
NS = 16
NW = NC * NS
LANES = 16
K = 80
CH_A = 148
CH_B = 104
CH_PAIR = CH_A + CH_B
E_PAD = NS * CH_PAIR * K
ROWS_PER_TILE = 624
ROWS_REM = N_NODES_C - ROWS_PER_TILE * NS


def _sc_partial_sums(src, dst, att, emb, zrows):
    mesh = plsc.VectorSubcoreMesh(
        core_axis_name="c", subcore_axis_name="s",
        num_cores=NC, num_subcores=NS)

    @functools.partial(
        pl.kernel,
        out_type=jax.ShapeDtypeStruct((NC, N_NODES_C, EMB_C), jnp.float32),
        mesh=mesh,
        compiler_params=pltpu.CompilerParams(use_tc_tiling_on_sc=False),
        scratch_types=[
            [pltpu.VMEM((K,), jnp.int32) for _ in range(4)],
            [pltpu.VMEM((K, EMB_C // 2), jnp.int32)
             for _ in range(4)],
            [pltpu.VMEM((K, EMB_C), jnp.float32) for _ in range(2)],
            [pltpu.VMEM((K,), jnp.int32) for _ in range(4)],
            [pltpu.VMEM((K,), jnp.float32) for _ in range(4)],
            pltpu.VMEM_SHARED((N_NODES_C, EMB_C), jnp.float32),
            [pltpu.SemaphoreType.DMA for _ in range(4)],
            [pltpu.SemaphoreType.DMA for _ in range(4)],
            [pltpu.SemaphoreType.DMA for _ in range(4)],
            [pltpu.SemaphoreType.DMA for _ in range(4)],
            [pltpu.SemaphoreType.DMA for _ in range(4)],
        ],
    )
    def body(src_hbm, dst_hbm, att_hbm, emb_hbm, z_hbm, out_hbm,
             srcr, bufs, fbufs, dstr, attr, acc_sh,
             gsem, ssem, dsem, asem, srsem):
        cid = lax.axis_index("c")
        sid = lax.axis_index("s")
        n_ch = jnp.where(cid == 0, CH_A, CH_B)
        chunk0 = sid * CH_PAIR + cid * CH_A

        row0 = sid * ROWS_PER_TILE
        pltpu.sync_copy(z_hbm.at[pl.ds(0, ROWS_PER_TILE)],
                        acc_sh.at[pl.ds(row0, ROWS_PER_TILE)])

        @pl.when(sid == 0)
        def _zero_rem():
            pltpu.sync_copy(
                z_hbm.at[pl.ds(0, ROWS_REM)],
                acc_sh.at[pl.ds(ROWS_PER_TILE * NS, ROWS_REM)])

        plsc.subcore_barrier()

        def gather(i, b):
            return pltpu.make_async_copy(
                emb_hbm.at[srcr[b]], bufs[b], gsem[b])

        def scatter_start(f, b):
            pltpu.async_copy(fbufs[f], acc_sh.at[dstr[b]], ssem[f], add=True)

        def scatter_wait(f, b):
            pltpu.make_async_copy(
                fbufs[f], acc_sh.at[dstr[b]], ssem[f]).wait()

        def src_copy(i, s):
            base = (chunk0 + i) * K
            return pltpu.make_async_copy(
                src_hbm.at[pl.ds(base, K)], srcr[s], srsem[s])

        def da_copies(i, s):
            base = (chunk0 + i) * K
            return (pltpu.make_async_copy(
                        dst_hbm.at[pl.ds(base, K)], dstr[s], dsem[s]),
                    pltpu.make_async_copy(
                        att_hbm.at[pl.ds(base, K)], attr[s], asem[s]))

        hi_mask = jnp.full((LANES,), -65536, jnp.int32)

        def scale(f, b):
            rows = bufs[b]
            out = fbufs[f]

            def group(g, c2):
                av = attr[b][pl.ds(g * LANES, LANES)]
                for j in range(LANES):
                    a = av[j]
                    e = g * LANES + j
                    for c in range(EMB_C // (2 * LANES)):
                        w = rows[e, pl.ds(c * LANES, LANES)]
                        lo = lax.bitcast_convert_type(
                            lax.shift_left(w, 16), jnp.float32)
                        hi = lax.bitcast_convert_type(w & hi_mask, jnp.float32)
                        out[e, pl.ds(c * LANES, LANES)] = lo * a
                        out[e, pl.ds(EMB_C // 2 + c * LANES, LANES)] = hi * a
                return c2
            lax.fori_loop(0, K // LANES, group, 0)

        src_copy(0, 0).start()
        src_copy(1, 1).start()
        src_copy(2, 2).start()
        src_copy(0, 0).wait()
        gather(0, 0).start()
        src_copy(1, 1).wait()
        gather(1, 1).start()
        for d in da_copies(0, 0):
            d.start()
        for d in da_copies(1, 1):
            d.start()

        def step(j, carry):
            for p in range(4):
                i = 4 * j + p
                b = p
                f = p % 2
                b2 = (p + 2) % 4
                b3 = (p + 3) % 4
                gather(i, b).wait()

                @pl.when(i >= 1)
                def _drain_prev():
                    scatter_wait((p + 1) % 2, b3)

                @pl.when(i + 3 < n_ch)
                def _src_pf():
                    src_copy(i + 3, b3).start()

                @pl.when(i + 2 < n_ch)
                def _next_gather():
                    src_copy(i + 2, b2).wait()
                    gather(i + 2, b2).start()
                    for d in da_copies(i + 2, b2):
                        d.start()
                for d in da_copies(i, b):
                    d.wait()
                scale(f, b)
                scatter_start(f, b)
            return carry
        lax.fori_loop(0, n_ch // 4, step, 0)
        scatter_wait(1, 3)

        plsc.subcore_barrier()
        pltpu.sync_copy(acc_sh.at[pl.ds(row0, ROWS_PER_TILE)],
                        out_hbm.at[cid, pl.ds(row0, ROWS_PER_TILE)])

        @pl.when(sid == 0)
        def _out_rem():
            pltpu.sync_copy(
                acc_sh.at[pl.ds(ROWS_PER_TILE * NS, ROWS_REM)],
                out_hbm.at[cid, pl.ds(ROWS_PER_TILE * NS, ROWS_REM)])

    return body(src, dst, att, emb, zrows)


def _tc_combine(parts):
    def body(a_ref, o_ref):
        o_ref[...] = a_ref[0] + a_ref[1]
    rows = 1000
    return pl.pallas_call(
        body,
        grid=(N_NODES_C // rows,),
        in_specs=[pl.BlockSpec((NC, rows, EMB_C), lambda i: (0, i, 0))],
        out_specs=pl.BlockSpec((rows, EMB_C), lambda i: (i, 0)),
        out_shape=jax.ShapeDtypeStruct((N_NODES_C, EMB_C), jnp.float32),
    )(parts)


@jax.jit
def kernel(edge_index, src_emb, e_att):
    pad = E_PAD - N_EDGES_C
    src = jnp.concatenate([edge_index[0], jnp.zeros((pad,), jnp.int32)])
    dst = jnp.concatenate([edge_index[1], jnp.zeros((pad,), jnp.int32)])
    att = jnp.concatenate([e_att.reshape(-1), jnp.zeros((pad,), jnp.float32)])
    cols = jnp.arange(EMB_C // 2, dtype=jnp.int32)
    perm = jnp.stack([cols, cols + EMB_C // 2], axis=1).reshape(-1)
    emb_bf = src_emb[:, perm].astype(jnp.bfloat16)
    emb_i32 = jax.lax.bitcast_convert_type(
        emb_bf.reshape(N_NODES_C, EMB_C // 2, 2), jnp.int32)
    zrows = jnp.zeros((ROWS_PER_TILE, EMB_C), jnp.float32)
    parts = _sc_partial_sums(src, dst, att, emb_i32, zrows)
    return _tc_combine(parts)


# speedup vs baseline: 1.7400x; 1.7400x over previous
import functools

import jax
import jax.numpy as jnp
from jax import lax
from jax.experimental import pallas as pl
from jax.experimental.pallas import tpu as pltpu
from jax.experimental.pallas import tpu_sc as plsc

N_NODES_C = 10000
N_EDGES_C = 320000
EMB_C = 128

NC = 2
NS = 16
NW = NC * NS
LANES = 16
K = 64
CH_A = 185
CH_B = 130
CH_PAIR = CH_A + CH_B
E_PAD = NS * CH_PAIR * K
ROWS_PER_TILE = 624
ROWS_REM = N_NODES_C - ROWS_PER_TILE * NS


def _sc_partial_sums(src, dst, att, emb, zrows):
    mesh = plsc.VectorSubcoreMesh(
        core_axis_name="c", subcore_axis_name="s",
        num_cores=NC, num_subcores=NS)

    @functools.partial(
        pl.kernel,
        out_type=jax.ShapeDtypeStruct((NC, N_NODES_C, EMB_C), jnp.float32),
        mesh=mesh,
        scratch_types=[
            [pltpu.VMEM((K,), jnp.int32) for _ in range(5)],
            [pltpu.VMEM((K, EMB_C), jnp.float32) for _ in range(5)],
            [pltpu.VMEM((K,), jnp.int32) for _ in range(5)],
            [pltpu.VMEM((K,), jnp.float32) for _ in range(5)],
            pltpu.VMEM_SHARED((N_NODES_C, EMB_C), jnp.float32),
            [pltpu.SemaphoreType.DMA for _ in range(5)],
            [pltpu.SemaphoreType.DMA for _ in range(5)],
            [pltpu.SemaphoreType.DMA for _ in range(5)],
            [pltpu.SemaphoreType.DMA for _ in range(5)],
            [pltpu.SemaphoreType.DMA for _ in range(5)],
        ],
    )
    def body(src_hbm, dst_hbm, att_hbm, emb_hbm, z_hbm, out_hbm,
             srcr, bufs, dstr, attr, acc_sh,
             gsem, ssem, dsem, asem, srsem):
        cid = lax.axis_index("c")
        sid = lax.axis_index("s")
        n_ch = jnp.where(cid == 0, CH_A, CH_B)
        chunk0 = sid * CH_PAIR + cid * CH_A

        row0 = sid * ROWS_PER_TILE
        pltpu.sync_copy(z_hbm.at[pl.ds(0, ROWS_PER_TILE)],
                        acc_sh.at[pl.ds(row0, ROWS_PER_TILE)])

        @pl.when(sid == 0)
        def _zero_rem():
            pltpu.sync_copy(
                z_hbm.at[pl.ds(0, ROWS_REM)],
                acc_sh.at[pl.ds(ROWS_PER_TILE * NS, ROWS_REM)])

        plsc.subcore_barrier()

        def gather(i, b):
            return pltpu.make_async_copy(
                emb_hbm.at[srcr[b]], bufs[b], gsem[b])

        def scatter_start(i, b):
            pltpu.async_copy(bufs[b], acc_sh.at[dstr[b]], ssem[b], add=True)

        def scatter_wait(i, b):
            pltpu.make_async_copy(bufs[b], acc_sh.at[dstr[b]], ssem[b]).wait()

        def src_copy(i, s):
            base = (chunk0 + i) * K
            return pltpu.make_async_copy(
                src_hbm.at[pl.ds(base, K)], srcr[s], srsem[s])

        def da_copies(i, s):
            base = (chunk0 + i) * K
            return (pltpu.make_async_copy(
                        dst_hbm.at[pl.ds(base, K)], dstr[s], dsem[s]),
                    pltpu.make_async_copy(
                        att_hbm.at[pl.ds(base, K)], attr[s], asem[s]))

        def scale(i, b):
            rows = bufs[b]

            def group(g, c2):
                av = attr[b][pl.ds(g * LANES, LANES)]
                for j in range(LANES):
                    a = av[j]
                    e = g * LANES + j
                    for c in range(EMB_C // LANES):
                        sl = pl.ds(c * LANES, LANES)
                        rows[e, sl] = rows[e, sl] * a
                return c2
            lax.fori_loop(0, K // LANES, group, 0)

        src_copy(0, 0).start()
        src_copy(1, 1).start()
        src_copy(2, 2).start()
        src_copy(3, 3).start()
        src_copy(0, 0).wait()
        gather(0, 0).start()
        src_copy(1, 1).wait()
        gather(1, 1).start()
        src_copy(2, 2).wait()
        gather(2, 2).start()
        for d in da_copies(0, 0):
            d.start()
        for d in da_copies(1, 1):
            d.start()

        def step(j, carry):
            for p in range(5):
                i = 5 * j + p
                b = p
                b2 = (p + 2) % 5
                b3 = (p + 3) % 5
                b4 = (p + 4) % 5
                gather(i, b).wait()

                @pl.when(i >= 2)
                def _drain_prev():
                    scatter_wait(i - 2, b3)

                @pl.when(i + 3 < n_ch)
                def _next_gather():
                    src_copy(i + 3, b3).wait()
                    gather(i + 3, b3).start()

                @pl.when(i + 4 < n_ch)
                def _src_pf():
                    src_copy(i + 4, b4).start()

                @pl.when(i + 2 < n_ch)
                def _next_da():
                    for d in da_copies(i + 2, b2):
                        d.start()
                for d in da_copies(i, b):
                    d.wait()
                scale(i, b)
                scatter_start(i, b)
            return carry
        lax.fori_loop(0, n_ch // 5, step, 0)
        scatter_wait(0, 3)
        scatter_wait(0, 4)

        plsc.subcore_barrier()
        pltpu.sync_copy(acc_sh.at[pl.ds(row0, ROWS_PER_TILE)],
                        out_hbm.at[cid, pl.ds(row0, ROWS_PER_TILE)])

        @pl.when(sid == 0)
        def _out_rem():
            pltpu.sync_copy(
                acc_sh.at[pl.ds(ROWS_PER_TILE * NS, ROWS_REM)],
                out_hbm.at[cid, pl.ds(ROWS_PER_TILE * NS, ROWS_REM)])

    return body(src, dst, att, emb, zrows)


def _tc_combine(parts):
    def body(a_ref, o_ref):
        o_ref[...] = a_ref[0] + a_ref[1]
    rows = 1000
    return pl.pallas_call(
        body,
        grid=(N_NODES_C // rows,),
        in_specs=[pl.BlockSpec((NC, rows, EMB_C), lambda i: (0, i, 0))],
        out_specs=pl.BlockSpec((rows, EMB_C), lambda i: (i, 0)),
        out_shape=jax.ShapeDtypeStruct((N_NODES_C, EMB_C), jnp.float32),
    )(parts)


@jax.jit
def kernel(edge_index, src_emb, e_att):
    pad = E_PAD - N_EDGES_C
    src = jnp.concatenate([edge_index[0], jnp.zeros((pad,), jnp.int32)])
    dst = jnp.concatenate([edge_index[1], jnp.zeros((pad,), jnp.int32)])
    att = jnp.concatenate([e_att.reshape(-1), jnp.zeros((pad,), jnp.float32)])
    zrows = jnp.zeros((ROWS_PER_TILE, EMB_C), jnp.float32)
    parts = _sc_partial_sums(src, dst, att, src_emb, zrows)
    return _tc_combine(parts)


# speedup vs baseline: 1.7649x; 1.0143x over previous
import functools

import jax
import jax.numpy as jnp
from jax import lax
from jax.experimental import pallas as pl
from jax.experimental.pallas import tpu as pltpu
from jax.experimental.pallas import tpu_sc as plsc

N_NODES_C = 10000
N_EDGES_C = 320000
EMB_C = 128

NC = 2
NS = 16
NW = NC * NS
LANES = 16
K = 64
CH_A = 210
CH_B = 105
CH_PAIR = CH_A + CH_B
E_PAD = NS * CH_PAIR * K
ROWS_PER_TILE = 624
ROWS_REM = N_NODES_C - ROWS_PER_TILE * NS


def _sc_partial_sums(src, dst, att, emb, zrows):
    mesh = plsc.VectorSubcoreMesh(
        core_axis_name="c", subcore_axis_name="s",
        num_cores=NC, num_subcores=NS)

    @functools.partial(
        pl.kernel,
        out_type=jax.ShapeDtypeStruct((NC, N_NODES_C, EMB_C), jnp.float32),
        mesh=mesh,
        scratch_types=[
            [pltpu.VMEM((K,), jnp.int32) for _ in range(5)],
            [pltpu.VMEM((K, EMB_C), jnp.float32) for _ in range(5)],
            [pltpu.VMEM((K,), jnp.int32) for _ in range(5)],
            [pltpu.VMEM((K,), jnp.float32) for _ in range(5)],
            pltpu.VMEM_SHARED((N_NODES_C, EMB_C), jnp.float32),
            [pltpu.SemaphoreType.DMA for _ in range(5)],
            [pltpu.SemaphoreType.DMA for _ in range(5)],
            [pltpu.SemaphoreType.DMA for _ in range(5)],
            [pltpu.SemaphoreType.DMA for _ in range(5)],
            [pltpu.SemaphoreType.DMA for _ in range(5)],
        ],
    )
    def body(src_hbm, dst_hbm, att_hbm, emb_hbm, z_hbm, out_hbm,
             srcr, bufs, dstr, attr, acc_sh,
             gsem, ssem, dsem, asem, srsem):
        cid = lax.axis_index("c")
        sid = lax.axis_index("s")
        n_ch = jnp.where(cid == 0, CH_A, CH_B)
        chunk0 = sid * CH_PAIR + cid * CH_A

        row0 = sid * ROWS_PER_TILE
        pltpu.sync_copy(z_hbm.at[pl.ds(0, ROWS_PER_TILE)],
                        acc_sh.at[pl.ds(row0, ROWS_PER_TILE)])

        @pl.when(sid == 0)
        def _zero_rem():
            pltpu.sync_copy(
                z_hbm.at[pl.ds(0, ROWS_REM)],
                acc_sh.at[pl.ds(ROWS_PER_TILE * NS, ROWS_REM)])

        plsc.subcore_barrier()

        def gather(i, b):
            return pltpu.make_async_copy(
                emb_hbm.at[srcr[b]], bufs[b], gsem[b])

        def scatter_start(i, b):
            pltpu.async_copy(bufs[b], acc_sh.at[dstr[b]], ssem[b], add=True)

        def scatter_wait(i, b):
            pltpu.make_async_copy(bufs[b], acc_sh.at[dstr[b]], ssem[b]).wait()

        def src_copy(i, s):
            base = (chunk0 + i) * K
            return pltpu.make_async_copy(
                src_hbm.at[pl.ds(base, K)], srcr[s], srsem[s])

        def da_copies(i, s):
            base = (chunk0 + i) * K
            return (pltpu.make_async_copy(
                        dst_hbm.at[pl.ds(base, K)], dstr[s], dsem[s]),
                    pltpu.make_async_copy(
                        att_hbm.at[pl.ds(base, K)], attr[s], asem[s]))

        def scale(i, b):
            rows = bufs[b]

            def group(g, c2):
                av = attr[b][pl.ds(g * LANES, LANES)]
                for j in range(LANES):
                    a = av[j]
                    e = g * LANES + j
                    for c in range(EMB_C // LANES):
                        sl = pl.ds(c * LANES, LANES)
                        rows[e, sl] = rows[e, sl] * a
                return c2
            lax.fori_loop(0, K // LANES, group, 0)

        src_copy(0, 0).start()
        src_copy(1, 1).start()
        src_copy(2, 2).start()
        src_copy(3, 3).start()
        src_copy(0, 0).wait()
        gather(0, 0).start()
        src_copy(1, 1).wait()
        gather(1, 1).start()
        src_copy(2, 2).wait()
        gather(2, 2).start()
        for d in da_copies(0, 0):
            d.start()
        for d in da_copies(1, 1):
            d.start()

        def step(j, carry):
            for p in range(5):
                i = 5 * j + p
                b = p
                b2 = (p + 2) % 5
                b3 = (p + 3) % 5
                b4 = (p + 4) % 5
                gather(i, b).wait()

                @pl.when(i >= 2)
                def _drain_prev():
                    scatter_wait(i - 2, b3)

                @pl.when(i + 3 < n_ch)
                def _next_gather():
                    src_copy(i + 3, b3).wait()
                    gather(i + 3, b3).start()

                @pl.when(i + 4 < n_ch)
                def _src_pf():
                    src_copy(i + 4, b4).start()

                @pl.when(i + 2 < n_ch)
                def _next_da():
                    for d in da_copies(i + 2, b2):
                        d.start()
                for d in da_copies(i, b):
                    d.wait()
                scale(i, b)
                scatter_start(i, b)
            return carry
        lax.fori_loop(0, n_ch // 5, step, 0)
        scatter_wait(0, 3)
        scatter_wait(0, 4)

        plsc.subcore_barrier()
        pltpu.sync_copy(acc_sh.at[pl.ds(row0, ROWS_PER_TILE)],
                        out_hbm.at[cid, pl.ds(row0, ROWS_PER_TILE)])

        @pl.when(sid == 0)
        def _out_rem():
            pltpu.sync_copy(
                acc_sh.at[pl.ds(ROWS_PER_TILE * NS, ROWS_REM)],
                out_hbm.at[cid, pl.ds(ROWS_PER_TILE * NS, ROWS_REM)])

    return body(src, dst, att, emb, zrows)


def _tc_combine(parts):
    def body(a_ref, o_ref):
        o_ref[...] = a_ref[0] + a_ref[1]
    rows = 1000
    return pl.pallas_call(
        body,
        grid=(N_NODES_C // rows,),
        in_specs=[pl.BlockSpec((NC, rows, EMB_C), lambda i: (0, i, 0))],
        out_specs=pl.BlockSpec((rows, EMB_C), lambda i: (i, 0)),
        out_shape=jax.ShapeDtypeStruct((N_NODES_C, EMB_C), jnp.float32),
    )(parts)


@jax.jit
def kernel(edge_index, src_emb, e_att):
    pad = E_PAD - N_EDGES_C
    src = jnp.concatenate([edge_index[0], jnp.zeros((pad,), jnp.int32)])
    dst = jnp.concatenate([edge_index[1], jnp.zeros((pad,), jnp.int32)])
    att = jnp.concatenate([e_att.reshape(-1), jnp.zeros((pad,), jnp.float32)])
    zrows = jnp.zeros((ROWS_PER_TILE, EMB_C), jnp.float32)
    parts = _sc_partial_sums(src, dst, att, src_emb, zrows)
    return _tc_combine(parts)
